# baseline (speedup 1.0000x reference)
import functools

import jax
import jax.numpy as jnp
from jax import lax
from jax.experimental import pallas as pl
from jax.experimental.pallas import tpu as pltpu
from jax.experimental.pallas import tpu_sc as plsc

EPS = 1e-5


def _ln_body(table_ref, gamma_ref, beta_ref, out_ref):
    x = table_ref[...]
    mean = jnp.mean(x, axis=-1, keepdims=True)
    xc = x - mean
    var = jnp.mean(xc * xc, axis=-1, keepdims=True)
    inv = lax.rsqrt(var + EPS)
    out_ref[...] = xc * inv * gamma_ref[...] + beta_ref[...]


def _ln_table(table, gamma, beta):
    v, d = table.shape
    block_rows = 10000
    assert v % block_rows == 0
    grid = (v // block_rows,)
    return pl.pallas_call(
        _ln_body,
        grid=grid,
        in_specs=[
            pl.BlockSpec((block_rows, d), lambda i: (i, 0)),
            pl.BlockSpec((1, d), lambda i: (0, 0)),
            pl.BlockSpec((1, d), lambda i: (0, 0)),
        ],
        out_specs=pl.BlockSpec((block_rows, d), lambda i: (i, 0)),
        out_shape=jax.ShapeDtypeStruct((v, d), jnp.float32),
    )(table, gamma.reshape(1, d), beta.reshape(1, d))


_GCHUNK = 128
_NBUF = 5
_PREF = 3


def _make_sc_gather(ntok, v, d):
    info = plsc.get_sparse_core_info()
    nw = info.num_cores * info.num_subcores
    assert ntok % (nw * _GCHUNK) == 0
    tw = ntok // nw
    ng = tw // _GCHUNK
    assert ng % _NBUF == 0
    mesh = plsc.VectorSubcoreMesh(core_axis_name="c", subcore_axis_name="s")

    @functools.partial(
        pl.kernel,
        mesh=mesh,
        out_type=jax.ShapeDtypeStruct((ntok, d), jnp.float32),
        scratch_types=[
            pltpu.VMEM((tw,), jnp.int32),
            pltpu.VMEM((_NBUF, _GCHUNK, d), jnp.float32),
            pltpu.SemaphoreType.DMA,
            pltpu.SemaphoreType.DMA,
        ],
    )
    def gather_kernel(tbl_hbm, ids_hbm, out_hbm, idx_v, rows_v, gsem, wsem):
        wid = lax.axis_index("s") * info.num_cores + lax.axis_index("c")
        base = wid * tw
        pltpu.sync_copy(ids_hbm.at[pl.ds(base, tw)], idx_v)

        def start_gather(g, b):
            pltpu.async_copy(
                tbl_hbm.at[idx_v.at[pl.ds(g * _GCHUNK, _GCHUNK)]],
                rows_v.at[b],
                gsem,
            )

        def start_write(g, b):
            pltpu.async_copy(
                rows_v.at[b],
                out_hbm.at[pl.ds(base + g * _GCHUNK, _GCHUNK)],
                wsem,
            )

        def wait_gather(b):
            pltpu.make_async_copy(tbl_hbm.at[idx_v.at[pl.ds(0, _GCHUNK)]],
                                  rows_v.at[b], gsem).wait()

        def wait_write(b):
            pltpu.make_async_copy(rows_v.at[b],
                                  out_hbm.at[pl.ds(base, _GCHUNK)], wsem).wait()

        for b in range(_PREF):
            start_gather(b, b)

        def outer(i, _):
            g0 = i * _NBUF
            for b in range(_NBUF):
                g = g0 + b
                wait_gather(b)
                start_write(g, b)
                j = g + _PREF
                bj = (b + _PREF) % _NBUF

                @pl.when(jnp.logical_and(j < ng, j >= _NBUF))
                def _():
                    wait_write(bj)
                    start_gather(j, bj)

                @pl.when(jnp.logical_and(j < ng, j < _NBUF))
                def _():
                    start_gather(j, bj)

            return 0

        lax.fori_loop(0, ng // _NBUF, outer, 0)
        for _ in range(min(_NBUF, ng)):
            wait_write(0)

    return gather_kernel


def kernel(input_ids, table, gamma, beta):
    b, s = input_ids.shape
    v, d = table.shape
    normed = _ln_table(table, gamma, beta)
    ids_flat = input_ids.reshape(-1).astype(jnp.int32)
    out = _make_sc_gather(b * s, v, d)(normed, ids_flat)
    return out.reshape(b, s, d)
